# baseline (speedup 1.0000x reference)
import jax
import jax.numpy as jnp
from jax.experimental import pallas as pl

_K = 16
_SIGMA = 0.0001

_noise_cache = {}


def _scaled_noise(b, n, m, dtype):
    ck = (b, n, m, jnp.dtype(dtype).name)
    if ck not in _noise_cache:
        nk = jax.random.fold_in(jax.random.key(0), 1)
        noise = jax.random.normal(nk, (b, n, 1, m), dtype=dtype)
        _noise_cache[ck] = jax.block_until_ready(
            (noise * _SIGMA).reshape(b * n, m))
    return _noise_cache[ck]


def _softtopk_kernel(x_ref, noise_ref, out_ref):
    v = x_ref[...] + noise_ref[...]
    m = v.shape[1]
    iota = jax.lax.broadcasted_iota(jnp.int32, v.shape, 1)
    idxs = []
    for _ in range(_K):
        minv = jnp.min(v, axis=1, keepdims=True)
        idx = jnp.min(jnp.where(v == minv, iota, m), axis=1, keepdims=True)
        idxs.append(idx)
        v = jnp.where(iota == idx, jnp.inf, v)
    for k in range(_K):
        out_ref[:, k, :] = (iota == idxs[k]).astype(jnp.float32)


def kernel(x):
    b, n, m = x.shape
    rows = b * n
    x2 = x.reshape(rows, m)
    noise = _scaled_noise(b, n, m, x.dtype)

    blk = 32 if rows % 32 == 0 else 1
    out = pl.pallas_call(
        _softtopk_kernel,
        grid=(rows // blk,),
        in_specs=[
            pl.BlockSpec((blk, m), lambda i: (i, 0)),
            pl.BlockSpec((blk, m), lambda i: (i, 0)),
        ],
        out_specs=pl.BlockSpec((blk, _K, m), lambda i: (i, 0, 0)),
        out_shape=jax.ShapeDtypeStruct((rows, _K, m), jnp.float32),
    )(x2, noise)
    return out.reshape(b, n, _K, m)
